# Initial kernel scaffold; baseline (speedup 1.0000x reference)
import jax
import jax.numpy as jnp
from jax.experimental import pallas as pl


def kernel(batch, weight):
    raise NotImplementedError("write your pallas kernel here")


# speedup vs baseline: 4.6777x; 4.6777x over previous
import functools

import jax
import jax.numpy as jnp
from jax import lax
from jax.experimental import pallas as pl
from jax.experimental.pallas import tpu as pltpu
from jax.experimental.pallas import tpu_sc as plsc

NC = 2
NS = 16
NW = NC * NS
CW = 128
NBUF = 5


@functools.cache
def _build(n_rows: int, vocab: int, dim: int):
    assert n_rows % (NW * CW) == 0
    ch = n_rows // (NW * CW)
    assert ch % NBUF == 0
    n_iter = ch // NBUF

    def body(table_hbm, idx_hbm, out_hbm, idx_v, *rest):
        bufs, sems = rest[:NBUF], rest[NBUF:]
        wid = lax.axis_index("s") * NC + lax.axis_index("c")
        row0 = wid * ch * CW

        pltpu.sync_copy(idx_hbm.at[wid], idx_v)

        for b in range(NBUF):
            pltpu.async_copy(table_hbm.at[idx_v.at[b]], bufs[b], sems[b])

        @pl.loop(0, n_iter - 1)
        def _(g):
            for b in range(NBUF):
                j = g * NBUF + b
                pltpu.make_async_copy(
                    table_hbm.at[idx_v.at[b]], bufs[b], sems[b]).wait()
                pltpu.sync_copy(
                    bufs[b], out_hbm.at[pl.ds(row0 + j * CW, CW)])
                pltpu.async_copy(
                    table_hbm.at[idx_v.at[j + NBUF]], bufs[b], sems[b])

        for b in range(NBUF):
            j = (n_iter - 1) * NBUF + b
            pltpu.make_async_copy(
                table_hbm.at[idx_v.at[b]], bufs[b], sems[b]).wait()
            pltpu.sync_copy(bufs[b], out_hbm.at[pl.ds(row0 + j * CW, CW)])

    return pl.kernel(
        body,
        out_type=jax.ShapeDtypeStruct((n_rows, dim), jnp.float32),
        mesh=plsc.VectorSubcoreMesh(core_axis_name="c", subcore_axis_name="s"),
        scratch_types=[
            pltpu.VMEM((ch, CW), jnp.int32),
            *[pltpu.VMEM((CW, dim), jnp.float32) for _ in range(NBUF)],
            *[pltpu.SemaphoreType.DMA for _ in range(NBUF)],
        ],
        compiler_params=pltpu.CompilerParams(use_tc_tiling_on_sc=False),
    )


def kernel(batch, weight):
    batch_sz, hist = batch.shape
    vocab, dim = weight.shape
    n_rows = batch_sz * hist
    idx3d = batch.reshape(NW, n_rows // (NW * CW), CW)
    out = _build(n_rows, vocab, dim)(weight, idx3d)
    return out.reshape(batch_sz, hist, dim)
